# Initial kernel scaffold; baseline (speedup 1.0000x reference)
import jax
import jax.numpy as jnp
from jax.experimental import pallas as pl


def kernel(x, edge_index, W1, b1, W2, b2, Wd, bd):
    raise NotImplementedError("write your pallas kernel here")


# speedup vs baseline: 6.9391x; 6.9391x over previous
import functools

import jax
import jax.numpy as jnp
from jax import lax
from jax.experimental import pallas as pl
from jax.experimental.pallas import tpu as pltpu
from jax.experimental.pallas import tpu_sc as plsc

_NC = 2
_NS = 16
_NW = _NC * _NS
_DEGW = 128
_EPS = 1e-7


def _plan(n, e):
    assert e % _NW == 0, e
    ew = e // _NW
    k = 0
    for cand in range(128, 0, -8):
        if ew % cand == 0:
            k = cand
            break
    assert k > 0, ew
    nb = ew // k
    zc = 0
    for cand in range(64, 0, -8):
        if n % cand == 0:
            zc = cand
            break
    assert zc > 0, n
    nzc = n // zc
    return ew, k, nb, zc, nzc


@functools.lru_cache(maxsize=None)
def _make_sc_aggr(n, d, e):
    ew, k, nb, zc, nzc = _plan(n, e)
    mesh = plsc.VectorSubcoreMesh(core_axis_name="c", subcore_axis_name="s")

    out_type = jax.ShapeDtypeStruct((_NC, n, d), jnp.float32)
    scratch = [
        pltpu.VMEM_SHARED((n, d), jnp.float32),
        pltpu.VMEM((1, nb, k), jnp.int32),
        pltpu.VMEM((1, nb, k), jnp.int32),
        pltpu.VMEM((k, d), jnp.float32),
        pltpu.VMEM((zc, d), jnp.float32),
        pltpu.SemaphoreType.DMA,
    ]

    def body(r_hbm, srcb_hbm, dstb_hbm, zer_hbm, aggr_out,
             aggr_sh, srcv, dstv, rows, zbuf, sem):
        c = lax.axis_index("c")
        s = lax.axis_index("s")
        wid = c * _NS + s

        pltpu.sync_copy(zer_hbm, zbuf)

        @pl.loop(s, nzc, step=_NS)
        def _zero(g):
            pltpu.sync_copy(zbuf, aggr_sh.at[pl.ds(g * zc, zc)])

        plsc.subcore_barrier()

        pltpu.sync_copy(srcb_hbm.at[pl.ds(wid, 1)], srcv)
        pltpu.sync_copy(dstb_hbm.at[pl.ds(wid, 1)], dstv)

        @pl.loop(0, nb)
        def _edges(j):
            pltpu.async_copy(r_hbm.at[srcv.at[0, j]], rows, sem).wait()
            pltpu.sync_copy(rows, aggr_sh.at[dstv.at[0, j]], add=True)

        plsc.subcore_barrier()

        @pl.loop(s, nzc, step=_NS)
        def _writeback(g):
            sl = pl.ds(g * zc, zc)
            pltpu.sync_copy(aggr_sh.at[sl], zbuf)
            pltpu.sync_copy(zbuf, aggr_out.at[c, sl])

    return pl.kernel(body, out_type=out_type, mesh=mesh,
                     scratch_types=scratch, name="sc_aggr")


@functools.lru_cache(maxsize=None)
def _make_sc_deg(n, e):
    ew, k, nb, zc, nzc = _plan(n, e)
    mesh = plsc.VectorSubcoreMesh(core_axis_name="c", subcore_axis_name="s")

    out_type = jax.ShapeDtypeStruct((_NC, n, _DEGW), jnp.float32)
    scratch = [
        pltpu.VMEM_SHARED((n, _DEGW), jnp.float32),
        pltpu.VMEM((1, nb, k), jnp.int32),
        pltpu.VMEM((k, _DEGW), jnp.float32),
        pltpu.VMEM((zc, _DEGW), jnp.float32),
        pltpu.SemaphoreType.DMA,
    ]

    def body(dstb_hbm, zdeg_hbm, ones_hbm, deg_out,
             deg_sh, dstv, onesv, degbuf, sem):
        c = lax.axis_index("c")
        s = lax.axis_index("s")
        wid = c * _NS + s

        pltpu.sync_copy(zdeg_hbm, degbuf)
        pltpu.sync_copy(ones_hbm, onesv)

        @pl.loop(s, nzc, step=_NS)
        def _zero(g):
            pltpu.sync_copy(degbuf, deg_sh.at[pl.ds(g * zc, zc)])

        plsc.subcore_barrier()

        pltpu.sync_copy(dstb_hbm.at[pl.ds(wid, 1)], dstv)

        @pl.loop(0, nb)
        def _edges(j):
            pltpu.sync_copy(onesv, deg_sh.at[dstv.at[0, j]], add=True)

        plsc.subcore_barrier()

        @pl.loop(s, nzc, step=_NS)
        def _writeback(g):
            sl = pl.ds(g * zc, zc)
            pltpu.sync_copy(deg_sh.at[sl], degbuf)
            pltpu.sync_copy(degbuf, deg_out.at[c, sl])

    return pl.kernel(body, out_type=out_type, mesh=mesh,
                     scratch_types=scratch, name="sc_deg")


def _tc_relu_eps(x_ref, out_ref):
    out_ref[...] = jnp.maximum(x_ref[...], 0.0) + _EPS


def _combine(p_ref, pdeg_ref):
    pdeg = pdeg_ref[...]
    deg = pdeg[0, :, 0:1] + pdeg[1, :, 0:1]
    rdeg = 1.0 / jnp.maximum(deg, 1.0)
    p = p_ref[...]
    return (p[0] + p[1]) * rdeg


def _tc_layer(h_ref, p_ref, pdeg_ref, w_ref, b_ref, outh_ref, outr_ref):
    a = _combine(p_ref, pdeg_ref)
    z = jnp.dot(h_ref[...] + a, w_ref[...],
                preferred_element_type=jnp.float32) + b_ref[...]
    zr = jnp.maximum(z, 0.0)
    outh_ref[...] = zr
    outr_ref[...] = zr + _EPS


def _tc_final(h_ref, p_ref, pdeg_ref, w_ref, b_ref, out_ref):
    a = _combine(p_ref, pdeg_ref)
    z = jnp.dot(h_ref[...] + a, w_ref[...],
                preferred_element_type=jnp.float32) + b_ref[...]
    nrm = jnp.sqrt(jnp.sum(z * z, axis=-1, keepdims=True))
    z = z / jnp.maximum(nrm, 1e-12)
    m = jnp.max(z, axis=-1, keepdims=True)
    lse = jnp.log(jnp.sum(jnp.exp(z - m), axis=-1, keepdims=True)) + m
    out_ref[...] = z - lse


def kernel(x, edge_index, W1, b1, W2, b2, Wd, bd):
    n, d = x.shape
    e = edge_index.shape[1]
    ew, k, nb, zc, nzc = _plan(n, e)

    src = edge_index[0].reshape(_NW, nb, k)
    dst = edge_index[1].reshape(_NW, nb, k)
    zer = jnp.zeros((zc, d), jnp.float32)
    zdeg = jnp.zeros((zc, _DEGW), jnp.float32)
    ones = jnp.ones((k, _DEGW), jnp.float32)

    nd = jax.ShapeDtypeStruct((n, d), jnp.float32)
    sc_aggr = _make_sc_aggr(n, d, e)
    sc_deg = _make_sc_deg(n, e)

    relu_eps = pl.pallas_call(_tc_relu_eps, out_shape=nd)
    layer = pl.pallas_call(_tc_layer, out_shape=(nd, nd))
    final = pl.pallas_call(_tc_final, out_shape=nd)

    b1r, b2r, bdr = (b.reshape(1, -1) for b in (b1, b2, bd))

    pdeg = sc_deg(dst, zdeg, ones)
    r1 = relu_eps(x)
    p1 = sc_aggr(r1, src, dst, zer)
    h1, r2 = layer(x, p1, pdeg, W1, b1r)
    p2 = sc_aggr(r2, src, dst, zer)
    h2, r3 = layer(h1, p2, pdeg, W2, b2r)
    p3 = sc_aggr(r3, src, dst, zer)
    return final(h2, p3, pdeg, Wd, bdr)


# speedup vs baseline: 8.6051x; 1.2401x over previous
import functools

import jax
import jax.numpy as jnp
from jax import lax
from jax.experimental import pallas as pl
from jax.experimental.pallas import tpu as pltpu
from jax.experimental.pallas import tpu_sc as plsc

_NC = 2
_NS = 16
_NW = _NC * _NS
_DEGW = 128
_EPS = 1e-7


def _plan(n, e):
    assert e % _NW == 0, e
    ew = e // _NW
    k = 0
    for cand in range(128, 0, -8):
        if ew % cand == 0:
            k = cand
            break
    assert k > 0, ew
    nb = ew // k
    zc = 0
    for cand in range(64, 0, -8):
        if n % cand == 0:
            zc = cand
            break
    assert zc > 0, n
    nzc = n // zc
    return ew, k, nb, zc, nzc


@functools.lru_cache(maxsize=None)
def _make_sc_aggr(n, d, e):
    ew, k, nb, zc_, nzc_ = _plan(n, e)
    assert nb % 2 == 1 and n % k == 0, (nb, k, n)
    zc, nzc = k, n // k
    mesh = plsc.VectorSubcoreMesh(core_axis_name="c", subcore_axis_name="s")

    out_type = jax.ShapeDtypeStruct((_NC, n, d), jnp.float32)
    scratch = [
        pltpu.VMEM_SHARED((n, d), jnp.float32),
        pltpu.VMEM((ew,), jnp.int32),
        pltpu.VMEM((1, nb, k), jnp.int32),
        pltpu.VMEM((k, d), jnp.float32),
        pltpu.VMEM((k, d), jnp.float32),
        pltpu.SemaphoreType.DMA,
        pltpu.SemaphoreType.DMA,
    ]

    def body(r_hbm, src1_hbm, dstb_hbm, zer_hbm, aggr_out,
             aggr_sh, srcv, dstv, rows_a, rows_b, sem_a, sem_b):
        c = lax.axis_index("c")
        s = lax.axis_index("s")
        wid = c * _NS + s

        def gather(j, rows, sem):
            pltpu.async_copy(r_hbm.at[srcv.at[pl.ds(j * k, k)]], rows, sem)

        def gwait(rows, sem):
            pltpu.make_async_copy(r_hbm.at[srcv.at[pl.ds(0, k)]], rows,
                                  sem).wait()

        def scatter(j, rows):
            pltpu.sync_copy(rows, aggr_sh.at[dstv.at[0, j]], add=True)

        pltpu.sync_copy(zer_hbm, rows_a)

        @pl.loop(s, nzc, step=_NS)
        def _zero(g):
            pltpu.sync_copy(rows_a, aggr_sh.at[pl.ds(g * zc, zc)])

        plsc.subcore_barrier()

        pltpu.sync_copy(src1_hbm.at[pl.ds(wid * ew, ew)], srcv)
        pltpu.sync_copy(dstb_hbm.at[pl.ds(wid, 1)], dstv)

        gather(0, rows_a, sem_a)

        @pl.loop(0, (nb - 1) // 2)
        def _edges(g):
            j0 = 2 * g
            gwait(rows_a, sem_a)
            gather(j0 + 1, rows_b, sem_b)
            scatter(j0, rows_a)
            gwait(rows_b, sem_b)
            gather(j0 + 2, rows_a, sem_a)
            scatter(j0 + 1, rows_b)

        gwait(rows_a, sem_a)
        scatter(nb - 1, rows_a)

        plsc.subcore_barrier()

        @pl.loop(s, nzc, step=_NS)
        def _writeback(g):
            sl = pl.ds(g * zc, zc)
            pltpu.sync_copy(aggr_sh.at[sl], rows_a)
            pltpu.sync_copy(rows_a, aggr_out.at[c, sl])

    return pl.kernel(body, out_type=out_type, mesh=mesh,
                     scratch_types=scratch, name="sc_aggr")


@functools.lru_cache(maxsize=None)
def _make_sc_deg(n, e):
    ew, k, nb, zc, nzc = _plan(n, e)
    mesh = plsc.VectorSubcoreMesh(core_axis_name="c", subcore_axis_name="s")

    out_type = jax.ShapeDtypeStruct((_NC, n, _DEGW), jnp.float32)
    scratch = [
        pltpu.VMEM_SHARED((n, _DEGW), jnp.float32),
        pltpu.VMEM((1, nb, k), jnp.int32),
        pltpu.VMEM((k, _DEGW), jnp.float32),
        pltpu.VMEM((zc, _DEGW), jnp.float32),
        pltpu.SemaphoreType.DMA,
    ]

    def body(dstb_hbm, zdeg_hbm, ones_hbm, deg_out,
             deg_sh, dstv, onesv, degbuf, sem):
        c = lax.axis_index("c")
        s = lax.axis_index("s")
        wid = c * _NS + s

        pltpu.sync_copy(zdeg_hbm, degbuf)
        pltpu.sync_copy(ones_hbm, onesv)

        @pl.loop(s, nzc, step=_NS)
        def _zero(g):
            pltpu.sync_copy(degbuf, deg_sh.at[pl.ds(g * zc, zc)])

        plsc.subcore_barrier()

        pltpu.sync_copy(dstb_hbm.at[pl.ds(wid, 1)], dstv)

        @pl.loop(0, nb)
        def _edges(j):
            pltpu.sync_copy(onesv, deg_sh.at[dstv.at[0, j]], add=True)

        plsc.subcore_barrier()

        @pl.loop(s, nzc, step=_NS)
        def _writeback(g):
            sl = pl.ds(g * zc, zc)
            pltpu.sync_copy(deg_sh.at[sl], degbuf)
            pltpu.sync_copy(degbuf, deg_out.at[c, sl])

    return pl.kernel(body, out_type=out_type, mesh=mesh,
                     scratch_types=scratch, name="sc_deg")


def _tc_relu_eps(x_ref, out_ref):
    out_ref[...] = jnp.maximum(x_ref[...], 0.0) + _EPS


def _combine(p_ref, pdeg_ref):
    pdeg = pdeg_ref[...]
    deg = pdeg[0, :, 0:1] + pdeg[1, :, 0:1]
    rdeg = 1.0 / jnp.maximum(deg, 1.0)
    p = p_ref[...]
    return (p[0] + p[1]) * rdeg


def _tc_layer(h_ref, p_ref, pdeg_ref, w_ref, b_ref, outh_ref, outr_ref):
    a = _combine(p_ref, pdeg_ref)
    z = jnp.dot(h_ref[...] + a, w_ref[...],
                preferred_element_type=jnp.float32) + b_ref[...]
    zr = jnp.maximum(z, 0.0)
    outh_ref[...] = zr
    outr_ref[...] = zr + _EPS


def _tc_final(h_ref, p_ref, pdeg_ref, w_ref, b_ref, out_ref):
    a = _combine(p_ref, pdeg_ref)
    z = jnp.dot(h_ref[...] + a, w_ref[...],
                preferred_element_type=jnp.float32) + b_ref[...]
    nrm = jnp.sqrt(jnp.sum(z * z, axis=-1, keepdims=True))
    z = z / jnp.maximum(nrm, 1e-12)
    m = jnp.max(z, axis=-1, keepdims=True)
    lse = jnp.log(jnp.sum(jnp.exp(z - m), axis=-1, keepdims=True)) + m
    out_ref[...] = z - lse


def kernel(x, edge_index, W1, b1, W2, b2, Wd, bd):
    n, d = x.shape
    e = edge_index.shape[1]
    ew, k, nb, zc, nzc = _plan(n, e)

    src = edge_index[0]
    dst = edge_index[1].reshape(_NW, nb, k)
    zer = jnp.zeros((k, d), jnp.float32)
    zdeg = jnp.zeros((zc, _DEGW), jnp.float32)
    ones = jnp.ones((k, _DEGW), jnp.float32)

    nd = jax.ShapeDtypeStruct((n, d), jnp.float32)
    sc_aggr = _make_sc_aggr(n, d, e)
    sc_deg = _make_sc_deg(n, e)

    relu_eps = pl.pallas_call(_tc_relu_eps, out_shape=nd)
    layer = pl.pallas_call(_tc_layer, out_shape=(nd, nd))
    final = pl.pallas_call(_tc_final, out_shape=nd)

    b1r, b2r, bdr = (b.reshape(1, -1) for b in (b1, b2, bd))

    pdeg = sc_deg(dst, zdeg, ones)
    r1 = relu_eps(x)
    p1 = sc_aggr(r1, src, dst, zer)
    h1, r2 = layer(x, p1, pdeg, W1, b1r)
    p2 = sc_aggr(r2, src, dst, zer)
    h2, r3 = layer(h1, p2, pdeg, W2, b2r)
    p3 = sc_aggr(r3, src, dst, zer)
    return final(h2, p3, pdeg, Wd, bdr)


# speedup vs baseline: 8.6061x; 1.0001x over previous
import functools

import jax
import jax.numpy as jnp
from jax import lax
from jax.experimental import pallas as pl
from jax.experimental.pallas import tpu as pltpu
from jax.experimental.pallas import tpu_sc as plsc

_NC = 2
_NS = 16
_NW = _NC * _NS
_DEGW = 128
_EPS = 1e-7


def _plan(n, e):
    assert e % _NW == 0, e
    ew = e // _NW
    k = 0
    for cand in range(128, 0, -8):
        if ew % cand == 0:
            k = cand
            break
    assert k > 0, ew
    nb = ew // k
    zc = 0
    for cand in range(64, 0, -8):
        if n % cand == 0:
            zc = cand
            break
    assert zc > 0, n
    nzc = n // zc
    return ew, k, nb, zc, nzc


@functools.lru_cache(maxsize=None)
def _make_sc_aggr(n, d, e):
    ew, k, nb, zc_, nzc_ = _plan(n, e)
    assert nb % 2 == 1 and n % k == 0, (nb, k, n)
    zc, nzc = k, n // k
    mesh = plsc.VectorSubcoreMesh(core_axis_name="c", subcore_axis_name="s")

    out_type = jax.ShapeDtypeStruct((_NC, n, d), jnp.float32)
    scratch = [
        pltpu.VMEM_SHARED((n, d), jnp.float32),
        pltpu.VMEM((ew,), jnp.int32),
        pltpu.VMEM((1, nb, k), jnp.int32),
        pltpu.VMEM((k, d), jnp.float32),
        pltpu.VMEM((k, d), jnp.float32),
        pltpu.SemaphoreType.DMA,
        pltpu.SemaphoreType.DMA,
        pltpu.SemaphoreType.DMA,
        pltpu.SemaphoreType.DMA,
    ]

    def body(r_hbm, src1_hbm, dstb_hbm, zer_hbm, aggr_out,
             aggr_sh, srcv, dstv, rows_a, rows_b,
             gsem_a, gsem_b, ssem_a, ssem_b):
        c = lax.axis_index("c")
        s = lax.axis_index("s")
        wid = c * _NS + s

        def gather(j, rows, sem):
            pltpu.async_copy(r_hbm.at[srcv.at[pl.ds(j * k, k)]], rows, sem)

        def gwait(rows, sem):
            pltpu.make_async_copy(r_hbm.at[srcv.at[pl.ds(0, k)]], rows,
                                  sem).wait()

        def scatter(j, rows, sem):
            pltpu.async_copy(rows, aggr_sh.at[dstv.at[0, j]], sem, add=True)

        def swait(rows, sem):
            pltpu.make_async_copy(zer_hbm, rows, sem).wait()

        pltpu.sync_copy(zer_hbm, rows_a)

        @pl.loop(s, nzc, step=_NS)
        def _zero(g):
            pltpu.sync_copy(rows_a, aggr_sh.at[pl.ds(g * zc, zc)])

        plsc.subcore_barrier()

        pltpu.sync_copy(src1_hbm.at[pl.ds(wid * ew, ew)], srcv)
        pltpu.sync_copy(dstb_hbm.at[pl.ds(wid, 1)], dstv)

        gather(0, rows_a, gsem_a)
        gwait(rows_a, gsem_a)
        scatter(0, rows_a, ssem_a)
        gather(1, rows_b, gsem_b)

        @pl.loop(0, (nb - 3) // 2)
        def _edges(g):
            j0 = 2 * g + 1
            gwait(rows_b, gsem_b)
            scatter(j0, rows_b, ssem_b)
            swait(rows_a, ssem_a)
            gather(j0 + 1, rows_a, gsem_a)
            gwait(rows_a, gsem_a)
            scatter(j0 + 1, rows_a, ssem_a)
            swait(rows_b, ssem_b)
            gather(j0 + 2, rows_b, gsem_b)

        gwait(rows_b, gsem_b)
        scatter(nb - 2, rows_b, ssem_b)
        swait(rows_a, ssem_a)
        gather(nb - 1, rows_a, gsem_a)
        gwait(rows_a, gsem_a)
        scatter(nb - 1, rows_a, ssem_a)
        swait(rows_b, ssem_b)
        swait(rows_a, ssem_a)

        plsc.subcore_barrier()

        @pl.loop(s, nzc, step=_NS)
        def _writeback(g):
            sl = pl.ds(g * zc, zc)
            pltpu.sync_copy(aggr_sh.at[sl], rows_a)
            pltpu.sync_copy(rows_a, aggr_out.at[c, sl])

    return pl.kernel(body, out_type=out_type, mesh=mesh,
                     scratch_types=scratch, name="sc_aggr")


@functools.lru_cache(maxsize=None)
def _make_sc_deg(n, e):
    ew, k, nb, zc, nzc = _plan(n, e)
    mesh = plsc.VectorSubcoreMesh(core_axis_name="c", subcore_axis_name="s")

    out_type = jax.ShapeDtypeStruct((_NC, n, _DEGW), jnp.float32)
    scratch = [
        pltpu.VMEM_SHARED((n, _DEGW), jnp.float32),
        pltpu.VMEM((1, nb, k), jnp.int32),
        pltpu.VMEM((k, _DEGW), jnp.float32),
        pltpu.VMEM((zc, _DEGW), jnp.float32),
        pltpu.SemaphoreType.DMA,
    ]

    def body(dstb_hbm, zdeg_hbm, ones_hbm, deg_out,
             deg_sh, dstv, onesv, degbuf, sem):
        c = lax.axis_index("c")
        s = lax.axis_index("s")
        wid = c * _NS + s

        pltpu.sync_copy(zdeg_hbm, degbuf)
        pltpu.sync_copy(ones_hbm, onesv)

        @pl.loop(s, nzc, step=_NS)
        def _zero(g):
            pltpu.sync_copy(degbuf, deg_sh.at[pl.ds(g * zc, zc)])

        plsc.subcore_barrier()

        pltpu.sync_copy(dstb_hbm.at[pl.ds(wid, 1)], dstv)

        win = 4

        @pl.loop(0, nb)
        def _edges(j):
            pltpu.async_copy(onesv, deg_sh.at[dstv.at[0, j]], sem, add=True)

            @pl.when(j >= win)
            def _drain():
                pltpu.make_async_copy(ones_hbm, onesv, sem).wait()

        for _ in range(win):
            pltpu.make_async_copy(ones_hbm, onesv, sem).wait()

        plsc.subcore_barrier()

        @pl.loop(s, nzc, step=_NS)
        def _writeback(g):
            sl = pl.ds(g * zc, zc)
            pltpu.sync_copy(deg_sh.at[sl], degbuf)
            pltpu.sync_copy(degbuf, deg_out.at[c, sl])

    return pl.kernel(body, out_type=out_type, mesh=mesh,
                     scratch_types=scratch, name="sc_deg")


def _tc_relu_eps(x_ref, out_ref):
    out_ref[...] = jnp.maximum(x_ref[...], 0.0) + _EPS


def _combine(p_ref, pdeg_ref):
    pdeg = pdeg_ref[...]
    deg = pdeg[0, :, 0:1] + pdeg[1, :, 0:1]
    rdeg = 1.0 / jnp.maximum(deg, 1.0)
    p = p_ref[...]
    return (p[0] + p[1]) * rdeg


def _tc_layer(h_ref, p_ref, pdeg_ref, w_ref, b_ref, outh_ref, outr_ref):
    a = _combine(p_ref, pdeg_ref)
    z = jnp.dot(h_ref[...] + a, w_ref[...],
                preferred_element_type=jnp.float32) + b_ref[...]
    zr = jnp.maximum(z, 0.0)
    outh_ref[...] = zr
    outr_ref[...] = zr + _EPS


def _tc_final(h_ref, p_ref, pdeg_ref, w_ref, b_ref, out_ref):
    a = _combine(p_ref, pdeg_ref)
    z = jnp.dot(h_ref[...] + a, w_ref[...],
                preferred_element_type=jnp.float32) + b_ref[...]
    nrm = jnp.sqrt(jnp.sum(z * z, axis=-1, keepdims=True))
    z = z / jnp.maximum(nrm, 1e-12)
    m = jnp.max(z, axis=-1, keepdims=True)
    lse = jnp.log(jnp.sum(jnp.exp(z - m), axis=-1, keepdims=True)) + m
    out_ref[...] = z - lse


def kernel(x, edge_index, W1, b1, W2, b2, Wd, bd):
    n, d = x.shape
    e = edge_index.shape[1]
    ew, k, nb, zc, nzc = _plan(n, e)

    src = edge_index[0]
    dst = edge_index[1].reshape(_NW, nb, k)
    zer = jnp.zeros((k, d), jnp.float32)
    zdeg = jnp.zeros((zc, _DEGW), jnp.float32)
    ones = jnp.ones((k, _DEGW), jnp.float32)

    nd = jax.ShapeDtypeStruct((n, d), jnp.float32)
    sc_aggr = _make_sc_aggr(n, d, e)
    sc_deg = _make_sc_deg(n, e)

    relu_eps = pl.pallas_call(_tc_relu_eps, out_shape=nd)
    layer = pl.pallas_call(_tc_layer, out_shape=(nd, nd))
    final = pl.pallas_call(_tc_final, out_shape=nd)

    b1r, b2r, bdr = (b.reshape(1, -1) for b in (b1, b2, bd))

    pdeg = sc_deg(dst, zdeg, ones)
    r1 = relu_eps(x)
    p1 = sc_aggr(r1, src, dst, zer)
    h1, r2 = layer(x, p1, pdeg, W1, b1r)
    p2 = sc_aggr(r2, src, dst, zer)
    h2, r3 = layer(h1, p2, pdeg, W2, b2r)
    p3 = sc_aggr(r3, src, dst, zer)
    return final(h2, p3, pdeg, Wd, bdr)
